# baseline (speedup 1.0000x reference)
import functools

import numpy as np
import jax
import jax.numpy as jnp
from jax import lax
from jax.experimental import pallas as pl
from jax.experimental.pallas import tpu as pltpu
from jax.experimental.pallas import tpu_sc as plsc

N = 10000
E = 320000
H = 16
NUM_BASIS = 10
MAX_RADIUS = 2.0
NUM_GRAPHS = 256
IN_CHANNELS = 100
FC_HIDDEN = 256

NUM_WORKERS = 32
EDGES_PER_WORKER = E // NUM_WORKERS
CHUNK = 2000
NCHUNKS = EDGES_PER_WORKER // CHUNK

BE = 6400
NB = N // 1000


def _sc_mesh():
    return plsc.VectorSubcoreMesh(core_axis_name="c", subcore_axis_name="s")


@functools.lru_cache(maxsize=None)
def _make_gather(num_tables):
    out_type = [jax.ShapeDtypeStruct((E, H), jnp.float32)
                for _ in range(num_tables)]

    @functools.partial(
        pl.kernel,
        mesh=_sc_mesh(),
        out_type=out_type,
        compiler_params=pltpu.CompilerParams(use_tc_tiling_on_sc=False),
        scratch_types=[
            pltpu.VMEM((CHUNK,), jnp.int32),
            pltpu.VMEM((CHUNK, H), jnp.float32),
            pltpu.SemaphoreType.DMA,
        ],
    )
    def gather_kernel(*refs):
        tables = refs[:num_tables]
        idx_hbm = refs[num_tables]
        outs = refs[num_tables + 1:num_tables + 1 + num_tables]
        idx_v, rows_v, sem = refs[num_tables + 1 + num_tables:]
        cid = lax.axis_index("c")
        sid = lax.axis_index("s")
        wid = cid * (NUM_WORKERS // 2) + sid
        base = wid * EDGES_PER_WORKER
        for c in range(NCHUNKS):
            off = base + c * CHUNK
            pltpu.sync_copy(idx_hbm.at[pl.ds(off, CHUNK)], idx_v)
            for t in range(num_tables):
                pltpu.async_copy(tables[t].at[idx_v], rows_v, sem).wait()
                pltpu.sync_copy(rows_v, outs[t].at[pl.ds(off, CHUNK)])

    return gather_kernel


@functools.lru_cache(maxsize=None)
def _make_scatter():
    @functools.partial(
        pl.kernel,
        mesh=_sc_mesh(),
        out_type=[jax.ShapeDtypeStruct((N, H), jnp.float32),
                  jax.ShapeDtypeStruct((N, H), jnp.float32)],
        compiler_params=pltpu.CompilerParams(use_tc_tiling_on_sc=False),
        scratch_types=[
            pltpu.VMEM((CHUNK,), jnp.int32),
            pltpu.VMEM((CHUNK, H), jnp.float32),
            pltpu.VMEM_SHARED((N, H), jnp.float32),
            pltpu.SemaphoreType.DMA,
        ],
    )
    def scatter_kernel(ef_hbm, dst_hbm, zeros_hbm, out_a, out_b,
                       idx_v, rows_v, acc, sem):
        cid = lax.axis_index("c")
        sid = lax.axis_index("s")
        wid = cid * (NUM_WORKERS // 2) + sid

        @pl.when(sid == 0)
        def _():
            pltpu.sync_copy(zeros_hbm, acc)

        plsc.subcore_barrier()

        base = wid * EDGES_PER_WORKER
        for c in range(NCHUNKS):
            off = base + c * CHUNK
            pltpu.sync_copy(dst_hbm.at[pl.ds(off, CHUNK)], idx_v)
            pltpu.sync_copy(ef_hbm.at[pl.ds(off, CHUNK)], rows_v)
            pltpu.sync_copy(rows_v, acc.at[idx_v], add=True)

        plsc.subcore_barrier()

        rows_out = N // 10

        @pl.when(sid < 10)
        def _():
            r0 = sid * rows_out

            @pl.when(cid == 0)
            def _():
                pltpu.sync_copy(acc.at[pl.ds(r0, rows_out)],
                                out_a.at[pl.ds(r0, rows_out)])

            @pl.when(cid == 1)
            def _():
                pltpu.sync_copy(acc.at[pl.ds(r0, rows_out)],
                                out_b.at[pl.ds(r0, rows_out)])

    return scatter_kernel


_BF = jnp.bfloat16


def _split(a):
    hi = a.astype(_BF)
    lo = (a - hi.astype(jnp.float32)).astype(_BF)
    return hi, lo


def _d(a, b):
    return jnp.dot(a, b, preferred_element_type=jnp.float32)


def _dot3(a, bhi, blo):
    ahi, alo = _split(a)
    return (_d(ahi, bhi) + _d(ahi, blo)) + _d(alo, bhi)


def _dot2(a, b_exact):
    ahi, alo = _split(a)
    return _d(ahi, b_exact) + _d(alo, b_exact)


def _edge_body(num_x, *refs):
    ea_ref = refs[0]
    xs = refs[1:1 + num_x]
    (w1h_ref, w1l_ref, w2h_ref, w2l_ref,
     rep_ref, sel_ref, out_ref) = refs[1 + num_x:]

    ea = ea_ref[...]
    r = jnp.sqrt(jnp.sum(ea * ea, axis=0, keepdims=True))
    step = MAX_RADIUS / (NUM_BASIS + 1)
    kk = (lax.broadcasted_iota(jnp.int32, (NUM_BASIS, 1), 0)
          + 1).astype(jnp.float32)
    diff = r / step - kk

    def sus(t):
        ts = jnp.where(t > 0.0, t, 1.0)
        return jnp.where(t > 0.0, jnp.exp(-1.0 / ts), 0.0)

    emb_t = sus(diff + 1.0) * sus(1.0 - diff)
    emb = jnp.transpose(emb_t)
    h1 = jnp.maximum(_dot3(emb, w1h_ref[...], w1l_ref[...]), 0.0)
    w = _dot3(h1, w2h_ref[...], w2l_ref[...])

    x1 = xs[0][...]
    for t in range(1, num_x):
        x1 = x1 + xs[t][...]
    xr = _dot2(x1, rep_ref[...])
    out_ref[...] = _dot2(w * xr, sel_ref[...])


def _make_edge_call(num_x):
    grid = (E // BE,)
    in_specs = (
        [pl.BlockSpec((3, BE), lambda i: (0, i))]
        + [pl.BlockSpec((BE, H), lambda i: (i, 0)) for _ in range(num_x)]
        + [
            pl.BlockSpec((NUM_BASIS, FC_HIDDEN), lambda i: (0, 0)),
            pl.BlockSpec((NUM_BASIS, FC_HIDDEN), lambda i: (0, 0)),
            pl.BlockSpec((FC_HIDDEN, H * H), lambda i: (0, 0)),
            pl.BlockSpec((FC_HIDDEN, H * H), lambda i: (0, 0)),
            pl.BlockSpec((H, H * H), lambda i: (0, 0)),
            pl.BlockSpec((H * H, H), lambda i: (0, 0)),
        ]
    )
    return pl.pallas_call(
        functools.partial(_edge_body, num_x),
        grid=grid,
        in_specs=in_specs,
        out_specs=pl.BlockSpec((BE, H), lambda i: (i, 0)),
        out_shape=jax.ShapeDtypeStruct((E, H), jnp.float32),
    )


_edge_call1 = _make_edge_call(1)
_edge_call2 = _make_edge_call(2)


def _embed_body(x_ref, table_ref, out_ref):
    xb = x_ref[...]
    classes = lax.broadcasted_iota(jnp.int32, (1, IN_CHANNELS), 1)
    onehot = (xb == classes).astype(_BF)
    th, tl = _split(table_ref[...])
    out_ref[...] = _d(onehot, th) + _d(onehot, tl)


_embed_call = pl.pallas_call(
    _embed_body,
    grid=(NB,),
    in_specs=[
        pl.BlockSpec((N // NB, 1), lambda i: (i, 0)),
        pl.BlockSpec((IN_CHANNELS, H), lambda i: (0, 0)),
    ],
    out_specs=pl.BlockSpec((N // NB, H), lambda i: (i, 0)),
    out_shape=jax.ShapeDtypeStruct((N, H), jnp.float32),
)


def _tail_body(pa_ref, pb_ref, b_ref, l0w_ref, l0b_ref, l1w_ref, l1b_ref,
               pw_ref, pbias_ref, out_ref, macc):
    i = pl.program_id(0)

    @pl.when(i == 0)
    def _():
        macc[...] = jnp.zeros_like(macc)

    h = pa_ref[...] + pb_ref[...]
    l0h, l0l = _split(l0w_ref[...])
    h = jnp.maximum(_dot3(h, l0h, l0l) + l0b_ref[...], 0.0)
    l1h, l1l = _split(l1w_ref[...])
    h = jnp.maximum(_dot3(h, l1h, l1l) + l1b_ref[...], 0.0)
    gids = lax.broadcasted_iota(jnp.int32, (1, NUM_GRAPHS), 1)
    onehot = (b_ref[...] == gids).astype(_BF)
    hh, hl = _split(h)
    seg = (lax.dot_general(onehot, hh, (((0,), (0,)), ((), ())),
                           preferred_element_type=jnp.float32)
           + lax.dot_general(onehot, hl, (((0,), (0,)), ((), ())),
                             preferred_element_type=jnp.float32))
    macc[...] += seg

    @pl.when(i == pl.num_programs(0) - 1)
    def _():
        pwh, pwl = _split(pw_ref[...])
        out_ref[...] = _dot3(macc[...], pwh, pwl) + pbias_ref[...]


_tail_call = pl.pallas_call(
    _tail_body,
    grid=(NB,),
    in_specs=[
        pl.BlockSpec((N // NB, H), lambda i: (i, 0)),
        pl.BlockSpec((N // NB, H), lambda i: (i, 0)),
        pl.BlockSpec((N // NB, 1), lambda i: (i, 0)),
        pl.BlockSpec((H, H), lambda i: (0, 0)),
        pl.BlockSpec((1, H), lambda i: (0, 0)),
        pl.BlockSpec((H, H), lambda i: (0, 0)),
        pl.BlockSpec((1, H), lambda i: (0, 0)),
        pl.BlockSpec((H, 1), lambda i: (0, 0)),
        pl.BlockSpec((1, 1), lambda i: (0, 0)),
    ],
    out_specs=pl.BlockSpec((NUM_GRAPHS, 1), lambda i: (0, 0)),
    out_shape=jax.ShapeDtypeStruct((NUM_GRAPHS, 1), jnp.float32),
    scratch_shapes=[pltpu.VMEM((NUM_GRAPHS, H), jnp.float32)],
)


_REP = np.repeat(np.eye(H, dtype=np.float32), H, axis=1)
_SEL = np.tile(np.eye(H, dtype=np.float32), (H, 1))
_OUT_SCALE = 1.0 / (np.sqrt(H) * np.sqrt(E / N))
_EMB_SCALE = 1.14136 * np.exp(2.0)


def kernel(x, edge_index, edge_attr, batch, embed,
           conv0_W1, conv0_W2, conv1_W1, conv1_W2,
           lin0_W, lin0_b, lin1_W, lin1_b, wprop_W, wprop_b):
    f32 = jnp.float32
    src = edge_index[0].astype(jnp.int32)
    dst = edge_index[1].astype(jnp.int32)
    x2 = x.astype(jnp.int32).reshape(N, 1)
    batch2 = batch.astype(jnp.int32).reshape(N, 1)

    w2_scale = np.sqrt(2.0) / np.sqrt(FC_HIDDEN) * _OUT_SCALE
    w1h_0, w1l_0 = _split((conv0_W1 * _EMB_SCALE).astype(f32))
    w1h_1, w1l_1 = _split((conv1_W1 * _EMB_SCALE).astype(f32))
    w2h_0, w2l_0 = _split((conv0_W2 * w2_scale).astype(f32))
    w2h_1, w2l_1 = _split((conv1_W2 * w2_scale).astype(f32))
    ea_t = edge_attr.T
    rep = jnp.asarray(_REP, dtype=_BF)
    sel = jnp.asarray(_SEL, dtype=_BF)
    zeros = jnp.zeros((N, H), f32)

    h0 = _embed_call(x2, embed.astype(f32))
    x1 = _make_gather(1)(h0, src)
    if isinstance(x1, (list, tuple)):
        x1 = x1[0]
    ef0 = _edge_call1(ea_t, x1, w1h_0, w1l_0, w2h_0, w2l_0, rep, sel)
    p0a, p0b = _make_scatter()(ef0, dst, zeros)
    x1a, x1b = _make_gather(2)(p0a, p0b, src)
    ef1 = _edge_call2(ea_t, x1a, x1b, w1h_1, w1l_1, w2h_1, w2l_1, rep, sel)
    p1a, p1b = _make_scatter()(ef1, dst, zeros)

    return _tail_call(p1a, p1b, batch2,
                      lin0_W.astype(f32), lin0_b.reshape(1, H).astype(f32),
                      lin1_W.astype(f32), lin1_b.reshape(1, H).astype(f32),
                      wprop_W.astype(f32), wprop_b.reshape(1, 1).astype(f32))


# speedup vs baseline: 1.0758x; 1.0758x over previous
import functools

import numpy as np
import jax
import jax.numpy as jnp
from jax import lax
from jax.experimental import pallas as pl
from jax.experimental.pallas import tpu as pltpu
from jax.experimental.pallas import tpu_sc as plsc

N = 10000
E = 320000
H = 16
NUM_BASIS = 10
MAX_RADIUS = 2.0
NUM_GRAPHS = 256
IN_CHANNELS = 100
FC_HIDDEN = 256

NUM_WORKERS = 32
EDGES_PER_WORKER = E // NUM_WORKERS
CHUNK = 2000
NCHUNKS = EDGES_PER_WORKER // CHUNK

BE = 6400
NB = N // 1000


def _sc_mesh():
    return plsc.VectorSubcoreMesh(core_axis_name="c", subcore_axis_name="s")


@functools.lru_cache(maxsize=None)
def _make_gather(num_tables):
    out_type = [jax.ShapeDtypeStruct((E, H), jnp.float32)
                for _ in range(num_tables)]

    @functools.partial(
        pl.kernel,
        mesh=_sc_mesh(),
        out_type=out_type,
        compiler_params=pltpu.CompilerParams(use_tc_tiling_on_sc=False),
        scratch_types=[
            pltpu.VMEM((CHUNK,), jnp.int32),
            pltpu.VMEM((CHUNK, H), jnp.float32),
            pltpu.SemaphoreType.DMA,
        ],
    )
    def gather_kernel(*refs):
        tables = refs[:num_tables]
        idx_hbm = refs[num_tables]
        outs = refs[num_tables + 1:num_tables + 1 + num_tables]
        idx_v, rows_v, sem = refs[num_tables + 1 + num_tables:]
        cid = lax.axis_index("c")
        sid = lax.axis_index("s")
        wid = cid * (NUM_WORKERS // 2) + sid
        base = wid * EDGES_PER_WORKER
        for c in range(NCHUNKS):
            off = base + c * CHUNK
            pltpu.sync_copy(idx_hbm.at[pl.ds(off, CHUNK)], idx_v)
            for t in range(num_tables):
                pltpu.async_copy(tables[t].at[idx_v], rows_v, sem).wait()
                pltpu.sync_copy(rows_v, outs[t].at[pl.ds(off, CHUNK)])

    return gather_kernel


@functools.lru_cache(maxsize=None)
def _make_scatter():
    @functools.partial(
        pl.kernel,
        mesh=_sc_mesh(),
        out_type=[jax.ShapeDtypeStruct((N, H), jnp.float32),
                  jax.ShapeDtypeStruct((N, H), jnp.float32)],
        compiler_params=pltpu.CompilerParams(use_tc_tiling_on_sc=False),
        scratch_types=[
            pltpu.VMEM((CHUNK,), jnp.int32),
            pltpu.VMEM((CHUNK, H), jnp.float32),
            pltpu.VMEM_SHARED((N, H), jnp.float32),
            pltpu.SemaphoreType.DMA,
        ],
    )
    def scatter_kernel(ef_hbm, dst_hbm, zeros_hbm, out_a, out_b,
                       idx_v, rows_v, acc, sem):
        cid = lax.axis_index("c")
        sid = lax.axis_index("s")
        wid = cid * (NUM_WORKERS // 2) + sid

        @pl.when(sid == 0)
        def _():
            pltpu.sync_copy(zeros_hbm, acc)

        plsc.subcore_barrier()

        base = wid * EDGES_PER_WORKER
        for c in range(NCHUNKS):
            off = base + c * CHUNK
            pltpu.sync_copy(dst_hbm.at[pl.ds(off, CHUNK)], idx_v)
            pltpu.sync_copy(ef_hbm.at[pl.ds(off, CHUNK)], rows_v)
            pltpu.sync_copy(rows_v, acc.at[idx_v], add=True)

        plsc.subcore_barrier()

        rows_out = N // 10

        @pl.when(sid < 10)
        def _():
            r0 = sid * rows_out

            @pl.when(cid == 0)
            def _():
                pltpu.sync_copy(acc.at[pl.ds(r0, rows_out)],
                                out_a.at[pl.ds(r0, rows_out)])

            @pl.when(cid == 1)
            def _():
                pltpu.sync_copy(acc.at[pl.ds(r0, rows_out)],
                                out_b.at[pl.ds(r0, rows_out)])

    return scatter_kernel


_BF = jnp.bfloat16


def _split(a):
    hi = a.astype(_BF)
    lo = (a - hi.astype(jnp.float32)).astype(_BF)
    return hi, lo


def _d(a, b):
    return jnp.dot(a, b, preferred_element_type=jnp.float32)


def _dot3(a, bhi, blo):
    ahi, alo = _split(a)
    return (_d(ahi, bhi) + _d(ahi, blo)) + _d(alo, bhi)


def _dot2(a, b_exact):
    ahi, alo = _split(a)
    return _d(ahi, b_exact) + _d(alo, b_exact)


def _edge_body(num_x, *refs):
    ea_ref = refs[0]
    xs = refs[1:1 + num_x]
    (w1h_ref, w1l_ref, w2h_ref, w2l_ref,
     ss_ref, rep_ref, sel_ref, out_ref) = refs[1 + num_x:]

    ea = ea_ref[...]
    r = jnp.sqrt(jnp.sum(ea * ea, axis=0, keepdims=True))
    step = MAX_RADIUS / (NUM_BASIS + 1)
    kk = (lax.broadcasted_iota(jnp.int32, (NUM_BASIS, 1), 0)
          + 1).astype(jnp.float32)
    diff = r / step - kk

    def sus(t):
        ts = jnp.where(t > 0.0, t, 1.0)
        return jnp.where(t > 0.0, jnp.exp(-1.0 / ts), 0.0)

    emb_t = sus(diff + 1.0) * sus(1.0 - diff)
    emb = jnp.transpose(emb_t)
    h1 = jnp.maximum(_dot3(emb, w1h_ref[...], w1l_ref[...]), 0.0)
    w = _dot3(h1, w2h_ref[...], w2l_ref[...])

    x1p = xs[0][...]
    for t in range(1, num_x):
        x1p = x1p + xs[t][...]
    xf = jnp.broadcast_to(x1p[:, None, :], (BE // 8, 8, 128)).reshape(BE, 128)
    ii = lax.broadcasted_iota(jnp.int32, (BE, 128), 0) % 8
    jj = lax.broadcasted_iota(jnp.int32, (BE, 128), 1) // H
    xm = jnp.where(ii == jj, xf, 0.0)
    x1 = _dot2(xm, ss_ref[...])
    xr = _dot2(x1, rep_ref[...])
    out_ref[...] = _dot2(w * xr, sel_ref[...])


def _make_edge_call(num_x):
    grid = (E // BE,)
    in_specs = (
        [pl.BlockSpec((3, BE), lambda i: (0, i))]
        + [pl.BlockSpec((BE // 8, 8 * H), lambda i: (i, 0))
           for _ in range(num_x)]
        + [
            pl.BlockSpec((NUM_BASIS, FC_HIDDEN), lambda i: (0, 0)),
            pl.BlockSpec((NUM_BASIS, FC_HIDDEN), lambda i: (0, 0)),
            pl.BlockSpec((FC_HIDDEN, H * H), lambda i: (0, 0)),
            pl.BlockSpec((FC_HIDDEN, H * H), lambda i: (0, 0)),
            pl.BlockSpec((128, H), lambda i: (0, 0)),
            pl.BlockSpec((H, H * H), lambda i: (0, 0)),
            pl.BlockSpec((H * H, H), lambda i: (0, 0)),
        ]
    )
    return pl.pallas_call(
        functools.partial(_edge_body, num_x),
        grid=grid,
        in_specs=in_specs,
        out_specs=pl.BlockSpec((BE, H), lambda i: (i, 0)),
        out_shape=jax.ShapeDtypeStruct((E, H), jnp.float32),
    )


_edge_call1 = _make_edge_call(1)
_edge_call2 = _make_edge_call(2)


def _embed_body(x_ref, table_ref, out_ref):
    xb = x_ref[...]
    classes = lax.broadcasted_iota(jnp.int32, (1, IN_CHANNELS), 1)
    onehot = (xb == classes).astype(_BF)
    th, tl = _split(table_ref[...])
    out_ref[...] = _d(onehot, th) + _d(onehot, tl)


_embed_call = pl.pallas_call(
    _embed_body,
    grid=(NB,),
    in_specs=[
        pl.BlockSpec((N // NB, 1), lambda i: (i, 0)),
        pl.BlockSpec((IN_CHANNELS, H), lambda i: (0, 0)),
    ],
    out_specs=pl.BlockSpec((N // NB, H), lambda i: (i, 0)),
    out_shape=jax.ShapeDtypeStruct((N, H), jnp.float32),
)


def _tail_body(pa_ref, pb_ref, b_ref, l0w_ref, l0b_ref, l1w_ref, l1b_ref,
               pw_ref, pbias_ref, out_ref, macc):
    i = pl.program_id(0)

    @pl.when(i == 0)
    def _():
        macc[...] = jnp.zeros_like(macc)

    h = pa_ref[...] + pb_ref[...]
    l0h, l0l = _split(l0w_ref[...])
    h = jnp.maximum(_dot3(h, l0h, l0l) + l0b_ref[...], 0.0)
    l1h, l1l = _split(l1w_ref[...])
    h = jnp.maximum(_dot3(h, l1h, l1l) + l1b_ref[...], 0.0)
    gids = lax.broadcasted_iota(jnp.int32, (1, NUM_GRAPHS), 1)
    onehot = (b_ref[...] == gids).astype(_BF)
    hh, hl = _split(h)
    seg = (lax.dot_general(onehot, hh, (((0,), (0,)), ((), ())),
                           preferred_element_type=jnp.float32)
           + lax.dot_general(onehot, hl, (((0,), (0,)), ((), ())),
                             preferred_element_type=jnp.float32))
    macc[...] += seg

    @pl.when(i == pl.num_programs(0) - 1)
    def _():
        pwh, pwl = _split(pw_ref[...])
        out_ref[...] = _dot3(macc[...], pwh, pwl) + pbias_ref[...]


_tail_call = pl.pallas_call(
    _tail_body,
    grid=(NB,),
    in_specs=[
        pl.BlockSpec((N // NB, H), lambda i: (i, 0)),
        pl.BlockSpec((N // NB, H), lambda i: (i, 0)),
        pl.BlockSpec((N // NB, 1), lambda i: (i, 0)),
        pl.BlockSpec((H, H), lambda i: (0, 0)),
        pl.BlockSpec((1, H), lambda i: (0, 0)),
        pl.BlockSpec((H, H), lambda i: (0, 0)),
        pl.BlockSpec((1, H), lambda i: (0, 0)),
        pl.BlockSpec((H, 1), lambda i: (0, 0)),
        pl.BlockSpec((1, 1), lambda i: (0, 0)),
    ],
    out_specs=pl.BlockSpec((NUM_GRAPHS, 1), lambda i: (0, 0)),
    out_shape=jax.ShapeDtypeStruct((NUM_GRAPHS, 1), jnp.float32),
    scratch_shapes=[pltpu.VMEM((NUM_GRAPHS, H), jnp.float32)],
)


_REP = np.repeat(np.eye(H, dtype=np.float32), H, axis=1)
_SEL = np.tile(np.eye(H, dtype=np.float32), (H, 1))
_SS = np.tile(np.eye(H, dtype=np.float32), (8, 1))
_OUT_SCALE = 1.0 / (np.sqrt(H) * np.sqrt(E / N))
_EMB_SCALE = 1.14136 * np.exp(2.0)


def kernel(x, edge_index, edge_attr, batch, embed,
           conv0_W1, conv0_W2, conv1_W1, conv1_W2,
           lin0_W, lin0_b, lin1_W, lin1_b, wprop_W, wprop_b):
    f32 = jnp.float32
    src = edge_index[0].astype(jnp.int32)
    dst = edge_index[1].astype(jnp.int32)
    x2 = x.astype(jnp.int32).reshape(N, 1)
    batch2 = batch.astype(jnp.int32).reshape(N, 1)

    w2_scale = np.sqrt(2.0) / np.sqrt(FC_HIDDEN) * _OUT_SCALE
    w1h_0, w1l_0 = _split((conv0_W1 * _EMB_SCALE).astype(f32))
    w1h_1, w1l_1 = _split((conv1_W1 * _EMB_SCALE).astype(f32))
    w2h_0, w2l_0 = _split((conv0_W2 * w2_scale).astype(f32))
    w2h_1, w2l_1 = _split((conv1_W2 * w2_scale).astype(f32))
    ea_t = edge_attr.T
    ss = jnp.asarray(_SS, dtype=_BF)
    rep = jnp.asarray(_REP, dtype=_BF)
    sel = jnp.asarray(_SEL, dtype=_BF)
    zeros = jnp.zeros((N, H), f32)

    def pack(a):
        return a.reshape(E // 8, 8 * H)

    h0 = _embed_call(x2, embed.astype(f32))
    x1 = _make_gather(1)(h0, src)
    if isinstance(x1, (list, tuple)):
        x1 = x1[0]
    ef0 = _edge_call1(ea_t, pack(x1), w1h_0, w1l_0, w2h_0, w2l_0,
                      ss, rep, sel)
    p0a, p0b = _make_scatter()(ef0, dst, zeros)
    x1a, x1b = _make_gather(2)(p0a, p0b, src)
    ef1 = _edge_call2(ea_t, pack(x1a), pack(x1b),
                      w1h_1, w1l_1, w2h_1, w2l_1, ss, rep, sel)
    p1a, p1b = _make_scatter()(ef1, dst, zeros)

    return _tail_call(p1a, p1b, batch2,
                      lin0_W.astype(f32), lin0_b.reshape(1, H).astype(f32),
                      lin1_W.astype(f32), lin1_b.reshape(1, H).astype(f32),
                      wprop_W.astype(f32), wprop_b.reshape(1, 1).astype(f32))
